# Initial kernel scaffold; baseline (speedup 1.0000x reference)
import jax
import jax.numpy as jnp
from jax.experimental import pallas as pl


def kernel(x, mask, alphas, W, b):
    raise NotImplementedError("write your pallas kernel here")


# speedup vs baseline: 1.2575x; 1.2575x over previous
import functools

import jax
import jax.numpy as jnp
from jax.experimental import pallas as pl
from jax.experimental.pallas import tpu as pltpu

NUM_OPS = 8
TM = 2048
TN = 512


def _body(x_ref, mask_ref, alphas_ref, w_ref, b_ref, o_ref):
    i = pl.program_id(2)

    a = alphas_ref[...]
    a = a - jnp.max(a)
    e = jnp.exp(a)
    p = e / jnp.sum(e)
    lane = jax.lax.broadcasted_iota(jnp.int32, (1, NUM_OPS), 1)
    p_i = jnp.sum(jnp.where(lane == i, p, 0.0))

    xm = x_ref[...] * mask_ref[...].astype(jnp.float32)
    acc = jnp.dot(xm, w_ref[0], preferred_element_type=jnp.float32)
    val = jnp.maximum(acc + b_ref[0], 0.0) * p_i

    @pl.when(i == 0)
    def _init():
        o_ref[...] = val

    @pl.when(i > 0)
    def _acc():
        o_ref[...] += val


@jax.jit
def kernel(x, mask, alphas, W, b):
    n_tok, d_model = x.shape
    num_ops = W.shape[0]
    mask2d = mask.reshape(n_tok, 1)
    alphas2d = alphas.reshape(1, num_ops)
    b3d = b.reshape(num_ops, 1, d_model)

    grid = (n_tok // TM, d_model // TN, num_ops)
    out = pl.pallas_call(
        _body,
        grid=grid,
        in_specs=[
            pl.BlockSpec((TM, d_model), lambda m, n, i: (m, 0)),
            pl.BlockSpec((TM, 1), lambda m, n, i: (m, 0)),
            pl.BlockSpec((1, num_ops), lambda m, n, i: (0, 0)),
            pl.BlockSpec((1, d_model, TN), lambda m, n, i: (i, 0, n)),
            pl.BlockSpec((1, 1, TN), lambda m, n, i: (i, 0, n)),
        ],
        out_specs=pl.BlockSpec((TM, TN), lambda m, n, i: (m, n)),
        out_shape=jax.ShapeDtypeStruct((n_tok, d_model), jnp.float32),
        compiler_params=pltpu.CompilerParams(
            dimension_semantics=("parallel", "parallel", "arbitrary"),
        ),
    )(x, mask2d, alphas2d, W, b3d)
    return out
